# Initial kernel scaffold; baseline (speedup 1.0000x reference)
import jax
import jax.numpy as jnp
from jax.experimental import pallas as pl


def kernel(x, edge_index, W1, b1, W2, b2, W3, b3):
    raise NotImplementedError("write your pallas kernel here")


# speedup vs baseline: 27.6194x; 27.6194x over previous
import functools

import jax
import jax.numpy as jnp
from jax import lax
from jax.experimental import pallas as pl
from jax.experimental.pallas import tpu as pltpu
from jax.experimental.pallas import tpu_sc as plsc

N = 100000
E = 1600000
N_PAD = 100096
EPAD = 1605632
E_ROWS = EPAD // 128
ROWS_PER_TILE = E_ROWS // 32
CHUNKS = ROWS_PER_TILE // 8
N_TILE = N_PAD // 16
ZB_ROWS = N_TILE // 8
BLK = 2000
GRID = N // BLK


def _mesh():
    return plsc.VectorSubcoreMesh(
        core_axis_name="c", subcore_axis_name="s", num_cores=2, num_subcores=16
    )


def _sc_degree(dstp):
    @functools.partial(
        pl.kernel,
        out_type=jax.ShapeDtypeStruct((2 * N_PAD, 16), jnp.float32),
        mesh=_mesh(),
        compiler_params=pltpu.CompilerParams(use_tc_tiling_on_sc=False),
        scratch_types=[
            pltpu.VMEM((8, 128), jnp.int32),
            pltpu.VMEM((128, 16), jnp.float32),
            pltpu.VMEM((ZB_ROWS, 16), jnp.float32),
            pltpu.VMEM_SHARED((N_PAD, 16), jnp.float32),
        ],
    )
    def k(dst_hbm, out_hbm, didx, ones_v, zb, acc):
        cid = lax.axis_index("c")
        sid = lax.axis_index("s")
        wid = sid * 2 + cid

        def initz(i, _):
            zb[i] = jnp.zeros((16,), jnp.float32)
            return 0

        lax.fori_loop(0, ZB_ROWS, initz, 0)

        def inito(i, _):
            ones_v[i] = jnp.ones((16,), jnp.float32)
            return 0

        lax.fori_loop(0, 128, inito, 0)
        base_n = sid * N_TILE

        def zcp(i, _):
            pltpu.sync_copy(zb, acc.at[pl.ds(base_n + i * ZB_ROWS, ZB_ROWS)])
            return 0

        lax.fori_loop(0, 8, zcp, 0)
        plsc.subcore_barrier()

        base = wid * ROWS_PER_TILE

        def chunk(g, _):
            pltpu.sync_copy(dst_hbm.at[pl.ds(base + g * 8, 8)], didx)
            for j in range(8):
                pltpu.sync_copy(ones_v, acc.at[didx.at[j]], add=True)
            return 0

        lax.fori_loop(0, CHUNKS, chunk, 0)
        plsc.subcore_barrier()
        pltpu.sync_copy(
            acc.at[pl.ds(base_n, N_TILE)],
            out_hbm.at[pl.ds(cid * N_PAD + base_n, N_TILE)],
        )

    return k(dstp)


def _sc_edge_agg(table, srcp, dstp):
    @functools.partial(
        pl.kernel,
        out_type=jax.ShapeDtypeStruct((2 * N_PAD, 16), jnp.float32),
        mesh=_mesh(),
        compiler_params=pltpu.CompilerParams(use_tc_tiling_on_sc=False),
        scratch_types=[
            pltpu.VMEM((8, 128), jnp.int32),
            pltpu.VMEM((8, 128), jnp.int32),
            pltpu.VMEM((1024, 16), jnp.float32),
            pltpu.VMEM((ZB_ROWS, 16), jnp.float32),
            pltpu.SemaphoreType.DMA,
            pltpu.VMEM_SHARED((N_PAD, 16), jnp.float32),
        ],
    )
    def k(table_hbm, src_hbm, dst_hbm, out_hbm, sidx, didx, rows, zb, sem, acc):
        cid = lax.axis_index("c")
        sid = lax.axis_index("s")
        wid = sid * 2 + cid

        def zrow(i, _):
            zb[i] = jnp.zeros((16,), jnp.float32)
            return 0

        lax.fori_loop(0, ZB_ROWS, zrow, 0)
        base_n = sid * N_TILE

        def zcp(i, _):
            pltpu.sync_copy(zb, acc.at[pl.ds(base_n + i * ZB_ROWS, ZB_ROWS)])
            return 0

        lax.fori_loop(0, 8, zcp, 0)
        plsc.subcore_barrier()

        base = wid * ROWS_PER_TILE

        def chunk(g, _):
            r0 = base + g * 8
            pltpu.sync_copy(src_hbm.at[pl.ds(r0, 8)], sidx)
            pltpu.sync_copy(dst_hbm.at[pl.ds(r0, 8)], didx)
            descs = [
                pltpu.async_copy(
                    table_hbm.at[sidx.at[j]], rows.at[pl.ds(j * 128, 128)], sem
                )
                for j in range(8)
            ]
            for j in range(8):
                descs[j].wait()
                pltpu.sync_copy(rows.at[pl.ds(j * 128, 128)], acc.at[didx.at[j]], add=True)
            return 0

        lax.fori_loop(0, CHUNKS, chunk, 0)
        plsc.subcore_barrier()
        pltpu.sync_copy(
            acc.at[pl.ds(base_n, N_TILE)],
            out_hbm.at[pl.ds(cid * N_PAD + base_n, N_TILE)],
        )

    return k(table, srcp, dstp)


def _tc_prep(x, W1, degs):
    def body(x_ref, w_ref, deg_ref, dinv_ref, h1p_ref):
        deg = deg_ref[0, :, 0:1] + deg_ref[1, :, 0:1] + 1.0
        dinv = lax.rsqrt(deg)
        w = w_ref[...]
        xb = x_ref[...]
        h = xb[:, 0:1] * w[0:1, :] + xb[:, 1:2] * w[1:2, :]
        dinv_ref[...] = dinv
        h1p_ref[...] = h * dinv

    return pl.pallas_call(
        body,
        grid=(GRID,),
        in_specs=[
            pl.BlockSpec((BLK, 2), lambda i: (i, 0)),
            pl.BlockSpec((2, 16), lambda i: (0, 0)),
            pl.BlockSpec((2, BLK, 16), lambda i: (0, i, 0)),
        ],
        out_specs=[
            pl.BlockSpec((BLK, 1), lambda i: (i, 0)),
            pl.BlockSpec((BLK, 16), lambda i: (i, 0)),
        ],
        out_shape=[
            jax.ShapeDtypeStruct((N, 1), jnp.float32),
            jax.ShapeDtypeStruct((N, 16), jnp.float32),
        ],
    )(x, W1, degs)


def _tc_mid1(a1, h1p, dinv, b1, W2):
    def body(a_ref, hp_ref, dinv_ref, b_ref, w_ref, oa_ref, ob_ref):
        dinv = dinv_ref[...]
        h = dinv * (a_ref[0] + a_ref[1] + hp_ref[...]) + b_ref[...]
        h = jnp.maximum(h, 0.0)
        h2 = jnp.dot(h, w_ref[...], preferred_element_type=jnp.float32) * dinv
        oa_ref[...] = h2[:, :16]
        ob_ref[...] = h2[:, 16:]

    return pl.pallas_call(
        body,
        grid=(GRID,),
        in_specs=[
            pl.BlockSpec((2, BLK, 16), lambda i: (0, i, 0)),
            pl.BlockSpec((BLK, 16), lambda i: (i, 0)),
            pl.BlockSpec((BLK, 1), lambda i: (i, 0)),
            pl.BlockSpec((1, 16), lambda i: (0, 0)),
            pl.BlockSpec((16, 32), lambda i: (0, 0)),
        ],
        out_specs=[
            pl.BlockSpec((BLK, 16), lambda i: (i, 0)),
            pl.BlockSpec((BLK, 16), lambda i: (i, 0)),
        ],
        out_shape=[
            jax.ShapeDtypeStruct((N, 16), jnp.float32),
            jax.ShapeDtypeStruct((N, 16), jnp.float32),
        ],
    )(a1, h1p, dinv, b1, W2)


def _tc_mid2(a2a, a2b, h2pa, h2pb, dinv, b2, W3):
    def body(aa_ref, ab_ref, hpa_ref, hpb_ref, dinv_ref, b_ref, w_ref, o_ref):
        dinv = dinv_ref[...]
        ha = aa_ref[0] + aa_ref[1] + hpa_ref[...]
        hb = ab_ref[0] + ab_ref[1] + hpb_ref[...]
        h2 = jnp.concatenate([ha, hb], axis=1)
        h2 = jnp.maximum(dinv * h2 + b_ref[...], 0.0)
        o_ref[...] = jnp.dot(h2, w_ref[...], preferred_element_type=jnp.float32) * dinv

    return pl.pallas_call(
        body,
        grid=(GRID,),
        in_specs=[
            pl.BlockSpec((2, BLK, 16), lambda i: (0, i, 0)),
            pl.BlockSpec((2, BLK, 16), lambda i: (0, i, 0)),
            pl.BlockSpec((BLK, 16), lambda i: (i, 0)),
            pl.BlockSpec((BLK, 16), lambda i: (i, 0)),
            pl.BlockSpec((BLK, 1), lambda i: (i, 0)),
            pl.BlockSpec((1, 32), lambda i: (0, 0)),
            pl.BlockSpec((32, 16), lambda i: (0, 0)),
        ],
        out_specs=pl.BlockSpec((BLK, 16), lambda i: (i, 0)),
        out_shape=jax.ShapeDtypeStruct((N, 16), jnp.float32),
    )(a2a, a2b, h2pa, h2pb, dinv, b2, W3)


def _tc_final(a3, h3p, dinv, b3):
    def body(a_ref, hp_ref, dinv_ref, b_ref, o_ref):
        o_ref[...] = dinv_ref[...] * (a_ref[0] + a_ref[1] + hp_ref[...]) + b_ref[...]

    return pl.pallas_call(
        body,
        grid=(GRID,),
        in_specs=[
            pl.BlockSpec((2, BLK, 16), lambda i: (0, i, 0)),
            pl.BlockSpec((BLK, 16), lambda i: (i, 0)),
            pl.BlockSpec((BLK, 1), lambda i: (i, 0)),
            pl.BlockSpec((1, 16), lambda i: (0, 0)),
        ],
        out_specs=pl.BlockSpec((BLK, 16), lambda i: (i, 0)),
        out_shape=jax.ShapeDtypeStruct((N, 16), jnp.float32),
    )(a3, h3p, dinv, b3)


def kernel(x, edge_index, W1, b1, W2, b2, W3, b3):
    ei = edge_index.astype(jnp.int32)
    pad = EPAD - E
    srcp = jnp.concatenate([ei[0], jnp.zeros((pad,), jnp.int32)]).reshape(E_ROWS, 128)
    dstp = jnp.concatenate([ei[1], jnp.full((pad,), N, jnp.int32)]).reshape(E_ROWS, 128)

    degs = _sc_degree(dstp).reshape(2, N_PAD, 16)
    dinv, h1p = _tc_prep(x, W1, degs)
    a1 = _sc_edge_agg(h1p, srcp, dstp).reshape(2, N_PAD, 16)
    h2pa, h2pb = _tc_mid1(a1, h1p, dinv, b1.reshape(1, 16), W2)
    a2a = _sc_edge_agg(h2pa, srcp, dstp).reshape(2, N_PAD, 16)
    a2b = _sc_edge_agg(h2pb, srcp, dstp).reshape(2, N_PAD, 16)
    h3p = _tc_mid2(a2a, a2b, h2pa, h2pb, dinv, b2.reshape(1, 32), W3)
    a3 = _sc_edge_agg(h3p, srcp, dstp).reshape(2, N_PAD, 16)
    return _tc_final(a3, h3p, dinv, b3.reshape(1, 16))


# speedup vs baseline: 38.4132x; 1.3908x over previous
import functools

import jax
import jax.numpy as jnp
from jax import lax
from jax.experimental import pallas as pl
from jax.experimental.pallas import tpu as pltpu
from jax.experimental.pallas import tpu_sc as plsc

N = 100000
E = 1600000
N_PAD = 100096
E_ROWS = E // 128
BASE_ROWS = 390
CHUNKS = 78
CB = 5
N_TILE = N_PAD // 16
ZB_ROWS = N_TILE // 16
PR = N_PAD // 8
BLKR = PR // 4
GRID = 4


def _mesh():
    return plsc.VectorSubcoreMesh(
        core_axis_name="c", subcore_axis_name="s", num_cores=2, num_subcores=16
    )


def _tile_base(wid):
    return wid * BASE_ROWS + jnp.minimum(wid, 20)


def _sc_degree(dstp):
    @functools.partial(
        pl.kernel,
        out_type=jax.ShapeDtypeStruct((2 * N_PAD, 16), jnp.float32),
        mesh=_mesh(),
        compiler_params=pltpu.CompilerParams(use_tc_tiling_on_sc=False),
        scratch_types=[
            pltpu.VMEM((CB, 128), jnp.int32),
            pltpu.VMEM((128, 16), jnp.float32),
            pltpu.VMEM((ZB_ROWS, 16), jnp.float32),
            pltpu.SemaphoreType.DMA,
            pltpu.VMEM_SHARED((N_PAD, 16), jnp.float32),
        ],
    )
    def k(dst_hbm, out_hbm, didx, ones_v, zb, sem, acc):
        cid = lax.axis_index("c")
        sid = lax.axis_index("s")
        wid = sid * 2 + cid

        def initz(i, _):
            zb[i] = jnp.zeros((16,), jnp.float32)
            return 0

        lax.fori_loop(0, ZB_ROWS, initz, 0)

        def inito(i, _):
            ones_v[i] = jnp.ones((16,), jnp.float32)
            return 0

        lax.fori_loop(0, 128, inito, 0)
        base_n = sid * N_TILE

        def zcp(i, _):
            pltpu.sync_copy(zb, acc.at[pl.ds(base_n + i * ZB_ROWS, ZB_ROWS)])
            return 0

        lax.fori_loop(0, 16, zcp, 0)
        plsc.subcore_barrier()

        base = _tile_base(wid)

        def chunk(g, _):
            pltpu.sync_copy(dst_hbm.at[pl.ds(base + g * CB, CB)], didx)
            for j in range(CB):
                pltpu.async_copy(ones_v, acc.at[didx.at[j]], sem, add=True)
            for j in range(CB):
                pltpu.make_async_copy(out_hbm.at[pl.ds(0, 128)], ones_v, sem).wait()
            return 0

        lax.fori_loop(0, CHUNKS, chunk, 0)

        @pl.when(wid < 20)
        def _():
            pltpu.sync_copy(dst_hbm.at[pl.ds(base + BASE_ROWS, 1)], didx.at[pl.ds(0, 1)])
            pltpu.sync_copy(ones_v, acc.at[didx.at[0]], add=True)

        plsc.subcore_barrier()
        pltpu.sync_copy(
            acc.at[pl.ds(base_n, N_TILE)],
            out_hbm.at[pl.ds(cid * N_PAD + base_n, N_TILE)],
        )

    return k(dstp)


def _sc_edge_agg(table, srcp, dstp):
    @functools.partial(
        pl.kernel,
        out_type=jax.ShapeDtypeStruct((2 * N_PAD, 16), jnp.float32),
        mesh=_mesh(),
        compiler_params=pltpu.CompilerParams(use_tc_tiling_on_sc=False),
        scratch_types=[
            pltpu.VMEM((CB, 128), jnp.int32),
            pltpu.VMEM((CB, 128), jnp.int32),
            pltpu.VMEM((CB, 128), jnp.int32),
            pltpu.VMEM((CB, 128), jnp.int32),
            pltpu.VMEM((CB * 128, 16), jnp.float32),
            pltpu.VMEM((CB * 128, 16), jnp.float32),
            pltpu.VMEM((ZB_ROWS, 16), jnp.float32),
            pltpu.SemaphoreType.DMA,
            pltpu.SemaphoreType.DMA,
            pltpu.SemaphoreType.DMA,
            pltpu.VMEM_SHARED((N_PAD, 16), jnp.float32),
        ],
    )
    def k(table_hbm, src_hbm, dst_hbm, out_hbm,
          sidxA, didxA, sidxB, didxB, rowsA, rowsB, zb, semI, semG, semS, acc):
        cid = lax.axis_index("c")
        sid = lax.axis_index("s")
        wid = sid * 2 + cid

        def zrow(i, _):
            zb[i] = jnp.zeros((16,), jnp.float32)
            return 0

        lax.fori_loop(0, ZB_ROWS, zrow, 0)
        base_n = sid * N_TILE

        def zcp(i, _):
            pltpu.sync_copy(zb, acc.at[pl.ds(base_n + i * ZB_ROWS, ZB_ROWS)])
            return 0

        lax.fori_loop(0, 16, zcp, 0)
        plsc.subcore_barrier()

        base = _tile_base(wid)

        def load_idx(c, sbuf, dbuf):
            r0 = base + c * CB
            pltpu.async_copy(src_hbm.at[pl.ds(r0, CB)], sbuf, semI)
            pltpu.async_copy(dst_hbm.at[pl.ds(r0, CB)], dbuf, semI)

        def drain_idx(sbuf, dbuf):
            pltpu.make_async_copy(src_hbm.at[pl.ds(0, CB)], sbuf, semI).wait()
            pltpu.make_async_copy(dst_hbm.at[pl.ds(0, CB)], dbuf, semI).wait()

        def fire_gathers(sbuf, rows):
            for j in range(CB):
                pltpu.async_copy(
                    table_hbm.at[sbuf.at[j]], rows.at[pl.ds(j * 128, 128)], semG
                )

        def drain_gathers(rows):
            for j in range(CB):
                pltpu.make_async_copy(
                    table_hbm.at[pl.ds(0, 128)], rows.at[pl.ds(j * 128, 128)], semG
                ).wait()

        def fire_scatters(dbuf, rows):
            for j in range(CB):
                pltpu.async_copy(
                    rows.at[pl.ds(j * 128, 128)], acc.at[dbuf.at[j]], semS, add=True
                )

        def drain_scatters(rows):
            for j in range(CB):
                pltpu.make_async_copy(
                    table_hbm.at[pl.ds(0, 128)], rows.at[pl.ds(j * 128, 128)], semS
                ).wait()

        load_idx(0, sidxA, didxA)
        load_idx(1, sidxB, didxB)

        def pair(g, _):
            cA = 2 * g
            drain_idx(sidxA, didxA)
            fire_gathers(sidxA, rowsA)
            drain_gathers(rowsA)
            fire_scatters(didxA, rowsA)
            drain_idx(sidxB, didxB)
            fire_gathers(sidxB, rowsB)
            drain_gathers(rowsB)
            drain_scatters(rowsA)
            load_idx(jnp.minimum(cA + 2, CHUNKS - 1), sidxA, didxA)
            fire_scatters(didxB, rowsB)
            drain_scatters(rowsB)
            load_idx(jnp.minimum(cA + 3, CHUNKS - 1), sidxB, didxB)
            return 0

        lax.fori_loop(0, CHUNKS // 2, pair, 0)

        drain_idx(sidxA, didxA)
        drain_idx(sidxB, didxB)

        @pl.when(wid < 20)
        def _():
            r = base + BASE_ROWS
            pltpu.sync_copy(src_hbm.at[pl.ds(r, 1)], sidxA.at[pl.ds(0, 1)])
            pltpu.sync_copy(dst_hbm.at[pl.ds(r, 1)], didxA.at[pl.ds(0, 1)])
            pltpu.async_copy(
                table_hbm.at[sidxA.at[0]], rowsA.at[pl.ds(0, 128)], semG
            ).wait()
            pltpu.sync_copy(rowsA.at[pl.ds(0, 128)], acc.at[didxA.at[0]], add=True)

        plsc.subcore_barrier()
        pltpu.sync_copy(
            acc.at[pl.ds(base_n, N_TILE)],
            out_hbm.at[pl.ds(cid * N_PAD + base_n, N_TILE)],
        )

    return k(table, srcp, dstp)


def _packed_specs(n_in):
    return [pl.BlockSpec((BLKR, 128), lambda i: (i, 0)) for _ in range(n_in)]


def _pair_spec():
    return pl.BlockSpec((2, BLKR, 128), lambda i: (0, i, 0))


def _w_spec():
    return pl.BlockSpec((128, 128), lambda i: (0, 0))


def _b_spec():
    return pl.BlockSpec((1, 128), lambda i: (0, 0))


def _packed_out(n=1):
    shape = jax.ShapeDtypeStruct((PR, 128), jnp.float32)
    spec = pl.BlockSpec((BLKR, 128), lambda i: (i, 0))
    if n == 1:
        return shape, spec
    return [shape] * n, [spec] * n


def _tc_matmul1(x16p, W1k):
    def body(x_ref, w_ref, h_ref):
        h_ref[...] = jnp.dot(x_ref[...], w_ref[...], preferred_element_type=jnp.float32)

    shape, spec = _packed_out()
    return pl.pallas_call(
        body,
        grid=(GRID,),
        in_specs=_packed_specs(1) + [_w_spec()],
        out_specs=spec,
        out_shape=shape,
    )(x16p, W1k)


def _tc_scale(h1p_, degp):
    def body(h_ref, deg_ref, dinv_ref, h1p_ref):
        dinv = lax.rsqrt(deg_ref[0] + deg_ref[1] + 1.0)
        dinv_ref[...] = dinv
        h1p_ref[...] = h_ref[...] * dinv

    shapes, specs = _packed_out(2)
    return pl.pallas_call(
        body,
        grid=(GRID,),
        in_specs=_packed_specs(1) + [_pair_spec()],
        out_specs=specs,
        out_shape=shapes,
    )(h1p_, degp)


def _tc_mid1(a1, h1p, dinv, b1t, W2A, W2B):
    def body(a_ref, hp_ref, dinv_ref, b_ref, wa_ref, wb_ref, oa_ref, ob_ref):
        dinv = dinv_ref[...]
        h = jnp.maximum(dinv * (a_ref[0] + a_ref[1] + hp_ref[...]) + b_ref[...], 0.0)
        oa_ref[...] = jnp.dot(h, wa_ref[...], preferred_element_type=jnp.float32) * dinv
        ob_ref[...] = jnp.dot(h, wb_ref[...], preferred_element_type=jnp.float32) * dinv

    shapes, specs = _packed_out(2)
    return pl.pallas_call(
        body,
        grid=(GRID,),
        in_specs=[_pair_spec()] + _packed_specs(2) + [_b_spec(), _w_spec(), _w_spec()],
        out_specs=specs,
        out_shape=shapes,
    )(a1, h1p, dinv, b1t, W2A, W2B)


def _tc_mid2(a2a, a2b, h2pa, h2pb, dinv, b2ta, b2tb, W3A, W3B):
    def body(aa_ref, ab_ref, hpa_ref, hpb_ref, dinv_ref, ba_ref, bb_ref,
             wa_ref, wb_ref, o_ref):
        dinv = dinv_ref[...]
        ha = jnp.maximum(dinv * (aa_ref[0] + aa_ref[1] + hpa_ref[...]) + ba_ref[...], 0.0)
        hb = jnp.maximum(dinv * (ab_ref[0] + ab_ref[1] + hpb_ref[...]) + bb_ref[...], 0.0)
        o_ref[...] = (
            jnp.dot(ha, wa_ref[...], preferred_element_type=jnp.float32)
            + jnp.dot(hb, wb_ref[...], preferred_element_type=jnp.float32)
        ) * dinv

    shape, spec = _packed_out()
    return pl.pallas_call(
        body,
        grid=(GRID,),
        in_specs=[_pair_spec(), _pair_spec()] + _packed_specs(3)
        + [_b_spec(), _b_spec(), _w_spec(), _w_spec()],
        out_specs=spec,
        out_shape=shape,
    )(a2a, a2b, h2pa, h2pb, dinv, b2ta, b2tb, W3A, W3B)


def _tc_final(a3, h3p, dinv, b3t):
    def body(a_ref, hp_ref, dinv_ref, b_ref, o_ref):
        o_ref[...] = dinv_ref[...] * (a_ref[0] + a_ref[1] + hp_ref[...]) + b_ref[...]

    shape, spec = _packed_out()
    return pl.pallas_call(
        body,
        grid=(GRID,),
        in_specs=[_pair_spec()] + _packed_specs(2) + [_b_spec()],
        out_specs=spec,
        out_shape=shape,
    )(a3, h3p, dinv, b3t)


def kernel(x, edge_index, W1, b1, W2, b2, W3, b3):
    ei = edge_index.astype(jnp.int32)
    srcp = ei[0].reshape(E_ROWS, 128)
    dstp = ei[1].reshape(E_ROWS, 128)

    eye8 = jnp.eye(8, dtype=jnp.float32)
    W1k = jnp.kron(eye8, jnp.tile(W1, (8, 1)) / 8.0)
    W2A = jnp.kron(eye8, W2[:, :16])
    W2B = jnp.kron(eye8, W2[:, 16:])
    W3A = jnp.kron(eye8, W3[:16, :])
    W3B = jnp.kron(eye8, W3[16:, :])
    b1t = jnp.tile(b1, 8).reshape(1, 128)
    b2ta = jnp.tile(b2[:16], 8).reshape(1, 128)
    b2tb = jnp.tile(b2[16:], 8).reshape(1, 128)
    b3t = jnp.tile(b3, 8).reshape(1, 128)
    x16p = jnp.tile(jnp.pad(x, ((0, N_PAD - N), (0, 0))), (1, 8)).reshape(PR, 128)

    h1 = _tc_matmul1(x16p, W1k)
    degp = _sc_degree(dstp).reshape(2, PR, 128)
    dinv, h1p = _tc_scale(h1, degp)
    h1p_t = h1p.reshape(N_PAD, 16)
    a1 = _sc_edge_agg(h1p_t, srcp, dstp).reshape(2, PR, 128)
    h2pa, h2pb = _tc_mid1(a1, h1p, dinv, b1t, W2A, W2B)
    a2a = _sc_edge_agg(h2pa.reshape(N_PAD, 16), srcp, dstp).reshape(2, PR, 128)
    a2b = _sc_edge_agg(h2pb.reshape(N_PAD, 16), srcp, dstp).reshape(2, PR, 128)
    h3p = _tc_mid2(a2a, a2b, h2pa, h2pb, dinv, b2ta, b2tb, W3A, W3B)
    a3 = _sc_edge_agg(h3p.reshape(N_PAD, 16), srcp, dstp).reshape(2, PR, 128)
    out = _tc_final(a3, h3p, dinv, b3t)
    return out.reshape(N_PAD, 16)[:N]


# speedup vs baseline: 38.4783x; 1.0017x over previous
import functools

import jax
import jax.numpy as jnp
from jax import lax
from jax.experimental import pallas as pl
from jax.experimental.pallas import tpu as pltpu
from jax.experimental.pallas import tpu_sc as plsc

N = 100000
E = 1600000
N_PAD = 100096
E_ROWS = E // 128
BASE_ROWS = 390
CHUNKS = 78
CB = 5
N_TILE = N_PAD // 16
ZB_ROWS = N_TILE // 16
PR = N_PAD // 8
BLKR = PR // 4
GRID = 4


def _mesh():
    return plsc.VectorSubcoreMesh(
        core_axis_name="c", subcore_axis_name="s", num_cores=2, num_subcores=16
    )


def _tile_base(wid):
    return wid * BASE_ROWS + jnp.minimum(wid, 20)


def _sc_degree(dstp):
    @functools.partial(
        pl.kernel,
        out_type=jax.ShapeDtypeStruct((2 * N_PAD, 16), jnp.float32),
        mesh=_mesh(),
        compiler_params=pltpu.CompilerParams(use_tc_tiling_on_sc=False),
        scratch_types=[
            pltpu.VMEM((CB, 128), jnp.int32),
            pltpu.VMEM((CB, 128), jnp.int32),
            pltpu.VMEM((128, 16), jnp.float32),
            pltpu.VMEM((ZB_ROWS, 16), jnp.float32),
            pltpu.SemaphoreType.DMA,
            pltpu.SemaphoreType.DMA,
            pltpu.VMEM_SHARED((N_PAD, 16), jnp.float32),
        ],
    )
    def k(dst_hbm, out_hbm, didxA, didxB, ones_v, zb, semI, semS, acc):
        cid = lax.axis_index("c")
        sid = lax.axis_index("s")
        wid = sid * 2 + cid

        def initz(i, _):
            zb[i] = jnp.zeros((16,), jnp.float32)
            return 0

        lax.fori_loop(0, ZB_ROWS, initz, 0)

        def inito(i, _):
            ones_v[i] = jnp.ones((16,), jnp.float32)
            return 0

        lax.fori_loop(0, 128, inito, 0)
        base_n = sid * N_TILE

        def zcp(i, _):
            pltpu.sync_copy(zb, acc.at[pl.ds(base_n + i * ZB_ROWS, ZB_ROWS)])
            return 0

        lax.fori_loop(0, 16, zcp, 0)
        plsc.subcore_barrier()

        base = _tile_base(wid)

        def load_idx(c, dbuf):
            pltpu.async_copy(dst_hbm.at[pl.ds(base + c * CB, CB)], dbuf, semI)

        def drain_idx(dbuf):
            pltpu.make_async_copy(dst_hbm.at[pl.ds(0, CB)], dbuf, semI).wait()

        def fire_scatters(dbuf):
            for j in range(CB):
                pltpu.async_copy(ones_v, acc.at[dbuf.at[j]], semS, add=True)

        def drain_scatters():
            for j in range(CB):
                pltpu.make_async_copy(out_hbm.at[pl.ds(0, 128)], ones_v, semS).wait()

        load_idx(0, didxA)
        load_idx(1, didxB)

        def pair(g, _):
            cA = 2 * g
            drain_idx(didxA)
            fire_scatters(didxA)
            drain_idx(didxB)
            fire_scatters(didxB)
            drain_scatters()
            load_idx(jnp.minimum(cA + 2, CHUNKS - 1), didxA)
            drain_scatters()
            load_idx(jnp.minimum(cA + 3, CHUNKS - 1), didxB)
            return 0

        lax.fori_loop(0, CHUNKS // 2, pair, 0)
        drain_idx(didxA)
        drain_idx(didxB)

        @pl.when(wid < 20)
        def _():
            pltpu.sync_copy(dst_hbm.at[pl.ds(base + BASE_ROWS, 1)], didxA.at[pl.ds(0, 1)])
            pltpu.sync_copy(ones_v, acc.at[didxA.at[0]], add=True)

        plsc.subcore_barrier()
        pltpu.sync_copy(
            acc.at[pl.ds(base_n, N_TILE)],
            out_hbm.at[pl.ds(cid * N_PAD + base_n, N_TILE)],
        )

    return k(dstp)


def _sc_edge_agg(table, srcp, dstp):
    @functools.partial(
        pl.kernel,
        out_type=jax.ShapeDtypeStruct((2 * N_PAD, 16), jnp.float32),
        mesh=_mesh(),
        compiler_params=pltpu.CompilerParams(use_tc_tiling_on_sc=False),
        scratch_types=[
            pltpu.VMEM((CB, 128), jnp.int32),
            pltpu.VMEM((CB, 128), jnp.int32),
            pltpu.VMEM((CB, 128), jnp.int32),
            pltpu.VMEM((CB, 128), jnp.int32),
            pltpu.VMEM((CB * 128, 16), jnp.float32),
            pltpu.VMEM((CB * 128, 16), jnp.float32),
            pltpu.VMEM((ZB_ROWS, 16), jnp.float32),
            pltpu.SemaphoreType.DMA,
            pltpu.SemaphoreType.DMA,
            pltpu.SemaphoreType.DMA,
            pltpu.VMEM_SHARED((N_PAD, 16), jnp.float32),
        ],
    )
    def k(table_hbm, src_hbm, dst_hbm, out_hbm,
          sidxA, didxA, sidxB, didxB, rowsA, rowsB, zb, semI, semG, semS, acc):
        cid = lax.axis_index("c")
        sid = lax.axis_index("s")
        wid = sid * 2 + cid

        def zrow(i, _):
            zb[i] = jnp.zeros((16,), jnp.float32)
            return 0

        lax.fori_loop(0, ZB_ROWS, zrow, 0)
        base_n = sid * N_TILE

        def zcp(i, _):
            pltpu.sync_copy(zb, acc.at[pl.ds(base_n + i * ZB_ROWS, ZB_ROWS)])
            return 0

        lax.fori_loop(0, 16, zcp, 0)
        plsc.subcore_barrier()

        base = _tile_base(wid)

        def load_idx(c, sbuf, dbuf):
            r0 = base + c * CB
            pltpu.async_copy(src_hbm.at[pl.ds(r0, CB)], sbuf, semI)
            pltpu.async_copy(dst_hbm.at[pl.ds(r0, CB)], dbuf, semI)

        def drain_idx(sbuf, dbuf):
            pltpu.make_async_copy(src_hbm.at[pl.ds(0, CB)], sbuf, semI).wait()
            pltpu.make_async_copy(dst_hbm.at[pl.ds(0, CB)], dbuf, semI).wait()

        def fire_gathers(sbuf, rows):
            for j in range(CB):
                pltpu.async_copy(
                    table_hbm.at[sbuf.at[j]], rows.at[pl.ds(j * 128, 128)], semG
                )

        def drain_gathers(rows):
            for j in range(CB):
                pltpu.make_async_copy(
                    table_hbm.at[pl.ds(0, 128)], rows.at[pl.ds(j * 128, 128)], semG
                ).wait()

        def fire_scatters(dbuf, rows):
            for j in range(CB):
                pltpu.async_copy(
                    rows.at[pl.ds(j * 128, 128)], acc.at[dbuf.at[j]], semS, add=True
                )

        def drain_scatters(rows):
            for j in range(CB):
                pltpu.make_async_copy(
                    table_hbm.at[pl.ds(0, 128)], rows.at[pl.ds(j * 128, 128)], semS
                ).wait()

        load_idx(0, sidxA, didxA)
        load_idx(1, sidxB, didxB)

        def pair(g, _):
            cA = 2 * g
            drain_idx(sidxA, didxA)
            fire_gathers(sidxA, rowsA)
            drain_gathers(rowsA)
            fire_scatters(didxA, rowsA)
            drain_idx(sidxB, didxB)
            fire_gathers(sidxB, rowsB)
            drain_gathers(rowsB)
            drain_scatters(rowsA)
            load_idx(jnp.minimum(cA + 2, CHUNKS - 1), sidxA, didxA)
            fire_scatters(didxB, rowsB)
            drain_scatters(rowsB)
            load_idx(jnp.minimum(cA + 3, CHUNKS - 1), sidxB, didxB)
            return 0

        lax.fori_loop(0, CHUNKS // 2, pair, 0)

        drain_idx(sidxA, didxA)
        drain_idx(sidxB, didxB)

        @pl.when(wid < 20)
        def _():
            r = base + BASE_ROWS
            pltpu.sync_copy(src_hbm.at[pl.ds(r, 1)], sidxA.at[pl.ds(0, 1)])
            pltpu.sync_copy(dst_hbm.at[pl.ds(r, 1)], didxA.at[pl.ds(0, 1)])
            pltpu.async_copy(
                table_hbm.at[sidxA.at[0]], rowsA.at[pl.ds(0, 128)], semG
            ).wait()
            pltpu.sync_copy(rowsA.at[pl.ds(0, 128)], acc.at[didxA.at[0]], add=True)

        plsc.subcore_barrier()
        pltpu.sync_copy(
            acc.at[pl.ds(base_n, N_TILE)],
            out_hbm.at[pl.ds(cid * N_PAD + base_n, N_TILE)],
        )

    return k(table, srcp, dstp)


def _packed_specs(n_in):
    return [pl.BlockSpec((BLKR, 128), lambda i: (i, 0)) for _ in range(n_in)]


def _pair_spec():
    return pl.BlockSpec((2, BLKR, 128), lambda i: (0, i, 0))


def _w_spec():
    return pl.BlockSpec((128, 128), lambda i: (0, 0))


def _b_spec():
    return pl.BlockSpec((1, 128), lambda i: (0, 0))


def _packed_out(n=1):
    shape = jax.ShapeDtypeStruct((PR, 128), jnp.float32)
    spec = pl.BlockSpec((BLKR, 128), lambda i: (i, 0))
    if n == 1:
        return shape, spec
    return [shape] * n, [spec] * n


def _tc_matmul1(x16p, W1k):
    def body(x_ref, w_ref, h_ref):
        h_ref[...] = jnp.dot(x_ref[...], w_ref[...], preferred_element_type=jnp.float32)

    shape, spec = _packed_out()
    return pl.pallas_call(
        body,
        grid=(GRID,),
        in_specs=_packed_specs(1) + [_w_spec()],
        out_specs=spec,
        out_shape=shape,
    )(x16p, W1k)


def _tc_scale(h1p_, degp):
    def body(h_ref, deg_ref, dinv_ref, h1p_ref):
        dinv = lax.rsqrt(deg_ref[0] + deg_ref[1] + 1.0)
        dinv_ref[...] = dinv
        h1p_ref[...] = h_ref[...] * dinv

    shapes, specs = _packed_out(2)
    return pl.pallas_call(
        body,
        grid=(GRID,),
        in_specs=_packed_specs(1) + [_pair_spec()],
        out_specs=specs,
        out_shape=shapes,
    )(h1p_, degp)


def _tc_mid1(a1, h1p, dinv, b1t, W2A, W2B):
    def body(a_ref, hp_ref, dinv_ref, b_ref, wa_ref, wb_ref, oa_ref, ob_ref):
        dinv = dinv_ref[...]
        h = jnp.maximum(dinv * (a_ref[0] + a_ref[1] + hp_ref[...]) + b_ref[...], 0.0)
        oa_ref[...] = jnp.dot(h, wa_ref[...], preferred_element_type=jnp.float32) * dinv
        ob_ref[...] = jnp.dot(h, wb_ref[...], preferred_element_type=jnp.float32) * dinv

    shapes, specs = _packed_out(2)
    return pl.pallas_call(
        body,
        grid=(GRID,),
        in_specs=[_pair_spec()] + _packed_specs(2) + [_b_spec(), _w_spec(), _w_spec()],
        out_specs=specs,
        out_shape=shapes,
    )(a1, h1p, dinv, b1t, W2A, W2B)


def _tc_mid2(a2a, a2b, h2pa, h2pb, dinv, b2ta, b2tb, W3A, W3B):
    def body(aa_ref, ab_ref, hpa_ref, hpb_ref, dinv_ref, ba_ref, bb_ref,
             wa_ref, wb_ref, o_ref):
        dinv = dinv_ref[...]
        ha = jnp.maximum(dinv * (aa_ref[0] + aa_ref[1] + hpa_ref[...]) + ba_ref[...], 0.0)
        hb = jnp.maximum(dinv * (ab_ref[0] + ab_ref[1] + hpb_ref[...]) + bb_ref[...], 0.0)
        o_ref[...] = (
            jnp.dot(ha, wa_ref[...], preferred_element_type=jnp.float32)
            + jnp.dot(hb, wb_ref[...], preferred_element_type=jnp.float32)
        ) * dinv

    shape, spec = _packed_out()
    return pl.pallas_call(
        body,
        grid=(GRID,),
        in_specs=[_pair_spec(), _pair_spec()] + _packed_specs(3)
        + [_b_spec(), _b_spec(), _w_spec(), _w_spec()],
        out_specs=spec,
        out_shape=shape,
    )(a2a, a2b, h2pa, h2pb, dinv, b2ta, b2tb, W3A, W3B)


def _tc_final(a3, h3p, dinv, b3t):
    def body(a_ref, hp_ref, dinv_ref, b_ref, o_ref):
        o_ref[...] = dinv_ref[...] * (a_ref[0] + a_ref[1] + hp_ref[...]) + b_ref[...]

    shape, spec = _packed_out()
    return pl.pallas_call(
        body,
        grid=(GRID,),
        in_specs=[_pair_spec()] + _packed_specs(2) + [_b_spec()],
        out_specs=spec,
        out_shape=shape,
    )(a3, h3p, dinv, b3t)


def kernel(x, edge_index, W1, b1, W2, b2, W3, b3):
    ei = edge_index.astype(jnp.int32)
    srcp = ei[0].reshape(E_ROWS, 128)
    dstp = ei[1].reshape(E_ROWS, 128)

    eye8 = jnp.eye(8, dtype=jnp.float32)
    W1k = jnp.kron(eye8, jnp.tile(W1, (8, 1)) / 8.0)
    W2A = jnp.kron(eye8, W2[:, :16])
    W2B = jnp.kron(eye8, W2[:, 16:])
    W3A = jnp.kron(eye8, W3[:16, :])
    W3B = jnp.kron(eye8, W3[16:, :])
    b1t = jnp.tile(b1, 8).reshape(1, 128)
    b2ta = jnp.tile(b2[:16], 8).reshape(1, 128)
    b2tb = jnp.tile(b2[16:], 8).reshape(1, 128)
    b3t = jnp.tile(b3, 8).reshape(1, 128)
    x16p = jnp.tile(jnp.pad(x, ((0, N_PAD - N), (0, 0))), (1, 8)).reshape(PR, 128)

    h1 = _tc_matmul1(x16p, W1k)
    degp = _sc_degree(dstp).reshape(2, PR, 128)
    dinv, h1p = _tc_scale(h1, degp)
    h1p_t = h1p.reshape(N_PAD, 16)
    a1 = _sc_edge_agg(h1p_t, srcp, dstp).reshape(2, PR, 128)
    h2pa, h2pb = _tc_mid1(a1, h1p, dinv, b1t, W2A, W2B)
    a2a = _sc_edge_agg(h2pa.reshape(N_PAD, 16), srcp, dstp).reshape(2, PR, 128)
    a2b = _sc_edge_agg(h2pb.reshape(N_PAD, 16), srcp, dstp).reshape(2, PR, 128)
    h3p = _tc_mid2(a2a, a2b, h2pa, h2pb, dinv, b2ta, b2tb, W3A, W3B)
    a3 = _sc_edge_agg(h3p.reshape(N_PAD, 16), srcp, dstp).reshape(2, PR, 128)
    out = _tc_final(a3, h3p, dinv, b3t)
    return out.reshape(N_PAD, 16)[:N]
